# baseline (speedup 1.0000x reference)
import jax
import jax.numpy as jnp
from jax.experimental import pallas as pl


def _update_body(x_ref, out_ref):
    out_ref[...] = x_ref[...] + 0.0


def kernel(x, _, edge_index, W1, b1, W2, b2):
    return pl.pallas_call(
        _update_body,
        out_shape=jax.ShapeDtypeStruct(x.shape, x.dtype),
    )(x)
